# baseline (speedup 1.0000x reference)
import jax
import jax.numpy as jnp
from jax import lax
from jax.experimental import pallas as pl
from jax.experimental.pallas import tpu as pltpu
from jax.experimental.pallas import tpu_sc as plsc

_H, _W, _KH, _KW = 768, 768, 3, 3
_K = _KH * _KW
_N = _H * _W * _K
_M = _H * _W
_NWORK = 32
_NPW = _N // _NWORK
_MPW = _M // _NWORK
_NBUF = 4
_C = 20736
_CM = _C // _K
_NCH = _NPW // _C
_L = 16
_GRP = _L * _K


def _sc_body(mask_hbm, s_hbm, out_hbm, *scr):
    s_bufs = scr[0:_NBUF]
    m_bufs = scr[_NBUF:2 * _NBUF]
    in_sems = scr[2 * _NBUF:3 * _NBUF]
    out_sems = scr[3 * _NBUF:4 * _NBUF]

    wid = lax.axis_index("s") * 2 + lax.axis_index("c")
    s_base = wid * _NPW
    m_base = wid * _MPW

    iota = lax.iota(jnp.int32, _L)
    idxk = [(iota + (_L * k)) // _K for k in range(_K)]

    def start_in(c):
        slot = c % _NBUF
        pltpu.make_async_copy(
            s_hbm.at[pl.ds(s_base + c * _C, _C)], s_bufs[slot], in_sems[slot]
        ).start()
        pltpu.make_async_copy(
            mask_hbm.at[pl.ds(m_base + c * _CM, _CM)], m_bufs[slot], in_sems[slot]
        ).start()

    def wait_in(c):
        slot = c % _NBUF
        pltpu.make_async_copy(
            s_hbm.at[pl.ds(s_base + c * _C, _C)], s_bufs[slot], in_sems[slot]
        ).wait()
        pltpu.make_async_copy(
            mask_hbm.at[pl.ds(m_base + c * _CM, _CM)], m_bufs[slot], in_sems[slot]
        ).wait()

    def start_out(c):
        slot = c % _NBUF
        pltpu.make_async_copy(
            s_bufs[slot], out_hbm.at[pl.ds(s_base + c * _C, _C)], out_sems[slot]
        ).start()

    def wait_out(c):
        slot = c % _NBUF
        pltpu.make_async_copy(
            s_bufs[slot], out_hbm.at[pl.ds(s_base + c * _C, _C)], out_sems[slot]
        ).wait()

    def compute(c):
        slot = c % _NBUF
        s_buf = s_bufs[slot]
        m_buf = m_bufs[slot]

        def group(g, carry):
            base = g * _GRP
            moff = jnp.broadcast_to(g * _L, (_L,))
            for k in range(_K):
                sv = s_buf[pl.ds(base + k * _L, _L)]
                mk = plsc.load_gather(m_buf, [moff + idxk[k]])
                s_buf[pl.ds(base + k * _L, _L)] = mk * sv
            return carry

        lax.fori_loop(0, _C // _GRP, group, 0)

    for c in range(_NBUF - 1):
        start_in(c)
    for c in range(_NCH):
        wait_in(c)
        compute(c)
        start_out(c)
        nxt = c + _NBUF - 1
        if nxt < _NCH:
            if c >= 1:
                wait_out(c - 1)
            start_in(nxt)
    for c in range(max(0, _NCH - _NBUF), _NCH):
        wait_out(c)


def kernel(mask, s_tensor):
    s_flat = s_tensor.reshape(_N)
    m_flat = mask.reshape(_M)
    mesh = plsc.VectorSubcoreMesh(core_axis_name="c", subcore_axis_name="s")
    out = pl.kernel(
        _sc_body,
        mesh=mesh,
        out_type=jax.ShapeDtypeStruct((_N,), jnp.float32),
        compiler_params=pltpu.CompilerParams(needs_layout_passes=False),
        scratch_types=(
            [pltpu.VMEM((_C,), jnp.float32) for _ in range(_NBUF)]
            + [pltpu.VMEM((_CM,), jnp.float32) for _ in range(_NBUF)]
            + [pltpu.SemaphoreType.DMA for _ in range(2 * _NBUF)]
        ),
    )(m_flat, s_flat)
    return out.reshape(_H, _W, _KH, _KW)


# speedup vs baseline: 269.2492x; 269.2492x over previous
import jax
import jax.numpy as jnp
from jax.experimental import pallas as pl

_H, _W, _KH, _KW = 768, 768, 3, 3
_P = _KH * _KW
_BR = 256


def _mul_body(m_ref, s_ref, o_ref):
    o_ref[...] = m_ref[...][None] * s_ref[...]


def kernel(mask, s_tensor):
    st = jnp.transpose(s_tensor, (2, 3, 0, 1)).reshape(_P, _H, _W)
    out = pl.pallas_call(
        _mul_body,
        grid=(_H // _BR, _P),
        in_specs=[
            pl.BlockSpec((_BR, _W), lambda r, p: (r, 0)),
            pl.BlockSpec((1, _BR, _W), lambda r, p: (p, r, 0)),
        ],
        out_specs=pl.BlockSpec((1, _BR, _W), lambda r, p: (p, r, 0)),
        out_shape=jax.ShapeDtypeStruct((_P, _H, _W), jnp.float32),
    )(mask, st)
    return out.reshape(_KH, _KW, _H, _W).transpose(2, 3, 0, 1)


# speedup vs baseline: 415.8350x; 1.5444x over previous
import jax
import jax.numpy as jnp
from jax.experimental import pallas as pl

_H, _W, _KH, _KW = 768, 768, 3, 3
_P = _KH * _KW
_BR = 256


def _mul_body(m_ref, s_ref, o_ref):
    o_ref[...] = m_ref[...][None] * s_ref[...]


def kernel(mask, s_tensor):
    st = jnp.transpose(s_tensor, (2, 3, 0, 1)).reshape(_P, _H, _W)
    out = pl.pallas_call(
        _mul_body,
        grid=(_P,),
        in_specs=[
            pl.BlockSpec((_H, _W), lambda p: (0, 0)),
            pl.BlockSpec((1, _H, _W), lambda p: (p, 0, 0)),
        ],
        out_specs=pl.BlockSpec((1, _H, _W), lambda p: (p, 0, 0)),
        out_shape=jax.ShapeDtypeStruct((_P, _H, _W), jnp.float32),
    )(mask, st)
    return out.reshape(_KH, _KW, _H, _W).transpose(2, 3, 0, 1)


# speedup vs baseline: 483.1655x; 1.1619x over previous
import jax
import jax.numpy as jnp
from jax.experimental import pallas as pl

_H, _W, _KH, _KW = 768, 768, 3, 3
_P = _KH * _KW
_BR = 256


def _mul_body(m_ref, s_ref, o_ref):
    o_ref[...] = m_ref[...][None] * s_ref[...]


def kernel(mask, s_tensor):
    st = jnp.transpose(s_tensor, (2, 3, 0, 1)).reshape(_P, _H, _W)
    out = pl.pallas_call(
        _mul_body,
        grid=(3,),
        in_specs=[
            pl.BlockSpec((_H, _W), lambda p: (0, 0)),
            pl.BlockSpec((3, _H, _W), lambda p: (p, 0, 0)),
        ],
        out_specs=pl.BlockSpec((3, _H, _W), lambda p: (p, 0, 0)),
        out_shape=jax.ShapeDtypeStruct((_P, _H, _W), jnp.float32),
    )(mask, st)
    return out.reshape(_KH, _KW, _H, _W).transpose(2, 3, 0, 1)
